# baseline (speedup 1.0000x reference)
import functools

import jax
import jax.numpy as jnp
from jax.experimental import pallas as pl
from jax.experimental.pallas import tpu as pltpu

_BLOCK = 8192
_LANES = 128
_IMAX = 2147483647


def _normalize(x, axis=1, eps=1e-12):
    n = jnp.linalg.norm(x, ord=2, axis=axis, keepdims=True)
    return x / jnp.maximum(n, eps)


def _fold_max(sims, width):
    m = sims[:, 0:width]
    for k in range(1, sims.shape[1] // width):
        m = jnp.maximum(m, sims[:, k * width:(k + 1) * width])
    return m


def _phase1_body(doc_ref, cls_ref, vmax_ref, menc_ref, cnt_ref,
                 amax_ref, aidx_ref, *, n_keys, n_blocks, block):
    g = pl.program_id(0)

    @pl.when(g == 0)
    def _init():
        amax_ref[...] = jnp.full(amax_ref.shape, -jnp.inf, jnp.float32)
        aidx_ref[...] = jnp.zeros(aidx_ref.shape, jnp.int32)

    doc = doc_ref[...]

    def _chunk_dot(k):
        ct = jnp.transpose(cls_ref[k * _LANES:(k + 1) * _LANES, :])
        return jax.lax.dot_general(
            doc, ct,
            dimension_numbers=(((1,), (0,)), ((), ())),
            preferred_element_type=jnp.float32)

    n_chunks = block // _LANES

    @pl.when(g < n_blocks - 1)
    def _update():
        m = _chunk_dot(0)
        for k in range(1, n_chunks):
            m = jnp.maximum(m, _chunk_dot(k))
        pred = m > amax_ref[...]
        amax_ref[...] = jnp.where(pred, m, amax_ref[...])
        aidx_ref[...] = jnp.where(pred, g, aidx_ref[...])

    @pl.when(g == n_blocks - 1)
    def _tail_and_finish():
        limit = n_keys - (n_blocks - 1) * block
        cols = jax.lax.broadcasted_iota(jnp.int32, (doc.shape[0], _LANES), 1)
        m = jnp.where(cols < limit, _chunk_dot(0), -jnp.inf)
        for k in range(1, n_chunks):
            mk = jnp.where(cols + k * _LANES < limit, _chunk_dot(k), -jnp.inf)
            m = jnp.maximum(m, mk)
        pred = m > amax_ref[...]
        amax = jnp.where(pred, m, amax_ref[...])
        aidx = jnp.where(pred, g, aidx_ref[...])
        vmax = jnp.max(amax, axis=1, keepdims=True)
        ach = amax == vmax
        lane = jax.lax.broadcasted_iota(jnp.int32, amax.shape, 1)
        enc = jnp.where(ach, aidx * _LANES + lane, _IMAX)
        vmax_ref[...] = vmax
        menc_ref[...] = jnp.min(enc, axis=1, keepdims=True)
        cnt_ref[...] = jnp.sum(ach.astype(jnp.int32), axis=1, keepdims=True)


def _phase2_body(doc_ref, gath_ref, vmax_ref, base_ref, idx_ref, ok_ref,
                 *, n_cand):
    sims2 = jax.lax.dot_general(
        doc_ref[...], gath_ref[...],
        dimension_numbers=(((1,), (0,)), ((), ())),
        preferred_element_type=jnp.float32)
    jrow = jax.lax.broadcasted_iota(jnp.int32, sims2.shape, 1)
    rrow = jax.lax.broadcasted_iota(jnp.int32, sims2.shape, 0)
    valid = (jrow // n_cand) == rrow
    sm = jnp.where(valid, sims2, -jnp.inf)
    mx2 = jnp.max(sm, axis=1, keepdims=True)
    jmin = jnp.min(jnp.where(sm == mx2, jrow, _IMAX), axis=1, keepdims=True)
    k = jnp.mod(jmin, n_cand)
    idx_ref[...] = base_ref[...] + _LANES * k
    ok_ref[...] = (mx2 == vmax_ref[...]).astype(jnp.int32)


def _full_body(doc_ref, cls_ref, val_ref, idx_ref, amax_ref, aidx_ref,
               *, n_keys, n_blocks, block):
    i = pl.program_id(0)

    @pl.when(i == 0)
    def _init():
        amax_ref[...] = jnp.full(amax_ref.shape, -jnp.inf, jnp.float32)
        aidx_ref[...] = jnp.zeros(aidx_ref.shape, jnp.int32)

    sims = jax.lax.dot_general(
        doc_ref[...], cls_ref[...],
        dimension_numbers=(((1,), (0,)), ((), ())),
        preferred_element_type=jnp.float32)

    @pl.when(i < n_blocks - 1)
    def _update():
        pred = sims > amax_ref[...]
        amax_ref[...] = jnp.where(pred, sims, amax_ref[...])
        aidx_ref[...] = jnp.where(pred, i, aidx_ref[...])

    @pl.when(i == n_blocks - 1)
    def _tail_and_finish():
        cols = jax.lax.broadcasted_iota(jnp.int32, sims.shape, 1)
        valid = cols < (n_keys - (n_blocks - 1) * block)
        last = jnp.where(valid, sims, -jnp.inf)
        pred = last > amax_ref[...]
        amax = jnp.where(pred, last, amax_ref[...])
        aidx = jnp.where(pred, i, aidx_ref[...])
        vmax = jnp.max(amax, axis=1, keepdims=True)
        gidx = aidx * block + cols
        cand = jnp.where(amax == vmax, gidx, _IMAX)
        val_ref[...] = vmax
        idx_ref[...] = jnp.min(cand, axis=1, keepdims=True)


def _full_kernel(doc_bf, cls_bf, n_keys, d, n_q, block, n_blocks):
    body = functools.partial(_full_body, n_keys=n_keys,
                             n_blocks=n_blocks, block=block)
    vals, idxs = pl.pallas_call(
        body,
        grid=(n_blocks,),
        in_specs=[
            pl.BlockSpec((n_q, d), lambda i: (0, 0)),
            pl.BlockSpec((d, block), lambda i: (0, i)),
        ],
        out_specs=[
            pl.BlockSpec((n_q, 1), lambda i: (0, 0)),
            pl.BlockSpec((n_q, 1), lambda i: (0, 0)),
        ],
        out_shape=[
            jax.ShapeDtypeStruct((n_q, 1), jnp.float32),
            jax.ShapeDtypeStruct((n_q, 1), jnp.int32),
        ],
        scratch_shapes=[
            pltpu.VMEM((n_q, block), jnp.float32),
            pltpu.VMEM((n_q, block), jnp.int32),
        ],
    )(doc_bf, cls_bf)
    return vals[:, 0], idxs[:, 0]


def kernel(doc_mtx, cls_mtx):
    n_keys, d = cls_mtx.shape
    n_q = doc_mtx.shape[0]
    block = _BLOCK
    n_blocks = pl.cdiv(n_keys, block)
    n_cand = block // _LANES

    doc_bf = _normalize(doc_mtx, axis=1).astype(jnp.bfloat16)
    cls_rows = _normalize(cls_mtx, axis=1).astype(jnp.bfloat16)

    p1 = functools.partial(_phase1_body, n_keys=n_keys,
                           n_blocks=n_blocks, block=block)
    vmax, menc, cnt = pl.pallas_call(
        p1,
        grid=(n_blocks,),
        in_specs=[
            pl.BlockSpec((n_q, d), lambda i: (0, 0)),
            pl.BlockSpec((block, d), lambda i: (i, 0)),
        ],
        out_specs=[
            pl.BlockSpec((n_q, 1), lambda i: (0, 0)),
            pl.BlockSpec((n_q, 1), lambda i: (0, 0)),
            pl.BlockSpec((n_q, 1), lambda i: (0, 0)),
        ],
        out_shape=[
            jax.ShapeDtypeStruct((n_q, 1), jnp.float32),
            jax.ShapeDtypeStruct((n_q, 1), jnp.int32),
            jax.ShapeDtypeStruct((n_q, 1), jnp.int32),
        ],
        scratch_shapes=[
            pltpu.VMEM((n_q, _LANES), jnp.float32),
            pltpu.VMEM((n_q, _LANES), jnp.int32),
        ],
    )(doc_bf, cls_rows)

    gwin = menc // _LANES
    cwin = jnp.mod(menc, _LANES)
    base = gwin * block + cwin
    cols = base + _LANES * jnp.arange(n_cand, dtype=jnp.int32)[None, :]
    cols = jnp.minimum(cols, n_keys - 1)
    gath = _normalize(jnp.take(cls_mtx, cols.reshape(-1), axis=0,
                               mode='clip')).astype(jnp.bfloat16).T

    qblk = min(128, n_q)
    p2 = functools.partial(_phase2_body, n_cand=n_cand)
    idx_fast, ok = pl.pallas_call(
        p2,
        grid=(n_q // qblk,),
        in_specs=[
            pl.BlockSpec((qblk, d), lambda j: (j, 0)),
            pl.BlockSpec((d, qblk * n_cand), lambda j: (0, j)),
            pl.BlockSpec((qblk, 1), lambda j: (j, 0)),
            pl.BlockSpec((qblk, 1), lambda j: (j, 0)),
        ],
        out_specs=[
            pl.BlockSpec((qblk, 1), lambda j: (j, 0)),
            pl.BlockSpec((qblk, 1), lambda j: (j, 0)),
        ],
        out_shape=[
            jax.ShapeDtypeStruct((n_q, 1), jnp.int32),
            jax.ShapeDtypeStruct((n_q, 1), jnp.int32),
        ],
    )(doc_bf, gath, vmax, base)

    need_fallback = (jnp.max(cnt) > 1) | (jnp.min(ok) < 1)

    def _slow(_):
        fb_block = 2048
        return _full_kernel(doc_bf, cls_rows.T, n_keys, d, n_q, fb_block,
                            pl.cdiv(n_keys, fb_block))

    def _fast(_):
        return vmax[:, 0], idx_fast[:, 0]

    return jax.lax.cond(need_fallback, _slow, _fast, None)


# speedup vs baseline: 1.2515x; 1.2515x over previous
import functools

import jax
import jax.numpy as jnp
from jax.experimental import pallas as pl
from jax.experimental.pallas import tpu as pltpu

_BLOCK = 8192
_LANES = 128
_IMAX = 2147483647


def _normalize(x, axis=1, eps=1e-12):
    n = jnp.linalg.norm(x, ord=2, axis=axis, keepdims=True)
    return x / jnp.maximum(n, eps)


def _fold_max(sims, width):
    m = sims[:, 0:width]
    for k in range(1, sims.shape[1] // width):
        m = jnp.maximum(m, sims[:, k * width:(k + 1) * width])
    return m


def _phase1_body(doc_ref, cls_ref, vmax_ref, menc_ref, cnt_ref,
                 amax_ref, aidx_ref, *, n_keys, n_blocks, block):
    g = pl.program_id(0)

    @pl.when(g == 0)
    def _init():
        amax_ref[...] = jnp.full(amax_ref.shape, -jnp.inf, jnp.float32)
        aidx_ref[...] = jnp.zeros(aidx_ref.shape, jnp.int32)

    doc = doc_ref[...]

    def _chunk_dot(k):
        return jax.lax.dot_general(
            doc, cls_ref[:, k * _LANES:(k + 1) * _LANES],
            dimension_numbers=(((1,), (0,)), ((), ())),
            preferred_element_type=jnp.float32)

    n_chunks = block // _LANES

    @pl.when(g < n_blocks - 1)
    def _update():
        ma = _chunk_dot(0)
        mb = _chunk_dot(1)
        for k in range(2, n_chunks, 2):
            ma = jnp.maximum(ma, _chunk_dot(k))
            mb = jnp.maximum(mb, _chunk_dot(k + 1))
        m = jnp.maximum(ma, mb)
        pred = m > amax_ref[...]
        amax_ref[...] = jnp.where(pred, m, amax_ref[...])
        aidx_ref[...] = jnp.where(pred, g, aidx_ref[...])

    @pl.when(g == n_blocks - 1)
    def _tail_and_finish():
        limit = n_keys - (n_blocks - 1) * block
        cols = jax.lax.broadcasted_iota(jnp.int32, (doc.shape[0], _LANES), 1)
        m = jnp.where(cols < limit, _chunk_dot(0), -jnp.inf)
        for k in range(1, n_chunks):
            mk = jnp.where(cols + k * _LANES < limit, _chunk_dot(k), -jnp.inf)
            m = jnp.maximum(m, mk)
        pred = m > amax_ref[...]
        amax = jnp.where(pred, m, amax_ref[...])
        aidx = jnp.where(pred, g, aidx_ref[...])
        vmax = jnp.max(amax, axis=1, keepdims=True)
        ach = amax == vmax
        lane = jax.lax.broadcasted_iota(jnp.int32, amax.shape, 1)
        enc = jnp.where(ach, aidx * _LANES + lane, _IMAX)
        vmax_ref[...] = vmax
        menc_ref[...] = jnp.min(enc, axis=1, keepdims=True)
        cnt_ref[...] = jnp.sum(ach.astype(jnp.int32), axis=1, keepdims=True)


def _phase2_body(doc_ref, gath_ref, vmax_ref, base_ref, idx_ref, ok_ref,
                 *, n_cand):
    sims2 = jax.lax.dot_general(
        doc_ref[...], gath_ref[...],
        dimension_numbers=(((1,), (0,)), ((), ())),
        preferred_element_type=jnp.float32)
    jrow = jax.lax.broadcasted_iota(jnp.int32, sims2.shape, 1)
    rrow = jax.lax.broadcasted_iota(jnp.int32, sims2.shape, 0)
    valid = (jrow // n_cand) == rrow
    sm = jnp.where(valid, sims2, -jnp.inf)
    mx2 = jnp.max(sm, axis=1, keepdims=True)
    jmin = jnp.min(jnp.where(sm == mx2, jrow, _IMAX), axis=1, keepdims=True)
    k = jnp.mod(jmin, n_cand)
    idx_ref[...] = base_ref[...] + _LANES * k
    ok_ref[...] = (mx2 == vmax_ref[...]).astype(jnp.int32)


def _full_body(doc_ref, cls_ref, val_ref, idx_ref, amax_ref, aidx_ref,
               *, n_keys, n_blocks, block):
    i = pl.program_id(0)

    @pl.when(i == 0)
    def _init():
        amax_ref[...] = jnp.full(amax_ref.shape, -jnp.inf, jnp.float32)
        aidx_ref[...] = jnp.zeros(aidx_ref.shape, jnp.int32)

    sims = jax.lax.dot_general(
        doc_ref[...], cls_ref[...],
        dimension_numbers=(((1,), (0,)), ((), ())),
        preferred_element_type=jnp.float32)

    @pl.when(i < n_blocks - 1)
    def _update():
        pred = sims > amax_ref[...]
        amax_ref[...] = jnp.where(pred, sims, amax_ref[...])
        aidx_ref[...] = jnp.where(pred, i, aidx_ref[...])

    @pl.when(i == n_blocks - 1)
    def _tail_and_finish():
        cols = jax.lax.broadcasted_iota(jnp.int32, sims.shape, 1)
        valid = cols < (n_keys - (n_blocks - 1) * block)
        last = jnp.where(valid, sims, -jnp.inf)
        pred = last > amax_ref[...]
        amax = jnp.where(pred, last, amax_ref[...])
        aidx = jnp.where(pred, i, aidx_ref[...])
        vmax = jnp.max(amax, axis=1, keepdims=True)
        gidx = aidx * block + cols
        cand = jnp.where(amax == vmax, gidx, _IMAX)
        val_ref[...] = vmax
        idx_ref[...] = jnp.min(cand, axis=1, keepdims=True)


def _full_kernel(doc_bf, cls_bf, n_keys, d, n_q, block, n_blocks):
    body = functools.partial(_full_body, n_keys=n_keys,
                             n_blocks=n_blocks, block=block)
    vals, idxs = pl.pallas_call(
        body,
        grid=(n_blocks,),
        in_specs=[
            pl.BlockSpec((n_q, d), lambda i: (0, 0)),
            pl.BlockSpec((d, block), lambda i: (0, i)),
        ],
        out_specs=[
            pl.BlockSpec((n_q, 1), lambda i: (0, 0)),
            pl.BlockSpec((n_q, 1), lambda i: (0, 0)),
        ],
        out_shape=[
            jax.ShapeDtypeStruct((n_q, 1), jnp.float32),
            jax.ShapeDtypeStruct((n_q, 1), jnp.int32),
        ],
        scratch_shapes=[
            pltpu.VMEM((n_q, block), jnp.float32),
            pltpu.VMEM((n_q, block), jnp.int32),
        ],
    )(doc_bf, cls_bf)
    return vals[:, 0], idxs[:, 0]


def kernel(doc_mtx, cls_mtx):
    n_keys, d = cls_mtx.shape
    n_q = doc_mtx.shape[0]
    block = _BLOCK
    n_blocks = pl.cdiv(n_keys, block)
    n_cand = block // _LANES

    doc_bf = _normalize(doc_mtx, axis=1).astype(jnp.bfloat16)
    cls_bf = _normalize(cls_mtx, axis=1).astype(jnp.bfloat16).T

    p1 = functools.partial(_phase1_body, n_keys=n_keys,
                           n_blocks=n_blocks, block=block)
    vmax, menc, cnt = pl.pallas_call(
        p1,
        grid=(n_blocks,),
        in_specs=[
            pl.BlockSpec((n_q, d), lambda i: (0, 0)),
            pl.BlockSpec((d, block), lambda i: (0, i)),
        ],
        out_specs=[
            pl.BlockSpec((n_q, 1), lambda i: (0, 0)),
            pl.BlockSpec((n_q, 1), lambda i: (0, 0)),
            pl.BlockSpec((n_q, 1), lambda i: (0, 0)),
        ],
        out_shape=[
            jax.ShapeDtypeStruct((n_q, 1), jnp.float32),
            jax.ShapeDtypeStruct((n_q, 1), jnp.int32),
            jax.ShapeDtypeStruct((n_q, 1), jnp.int32),
        ],
        scratch_shapes=[
            pltpu.VMEM((n_q, _LANES), jnp.float32),
            pltpu.VMEM((n_q, _LANES), jnp.int32),
        ],
    )(doc_bf, cls_bf)

    gwin = menc // _LANES
    cwin = jnp.mod(menc, _LANES)
    base = gwin * block + cwin
    cols = base + _LANES * jnp.arange(n_cand, dtype=jnp.int32)[None, :]
    cols = jnp.minimum(cols, n_keys - 1)
    gath = _normalize(jnp.take(cls_mtx, cols.reshape(-1), axis=0,
                               mode='clip')).astype(jnp.bfloat16).T

    qblk = min(128, n_q)
    p2 = functools.partial(_phase2_body, n_cand=n_cand)
    idx_fast, ok = pl.pallas_call(
        p2,
        grid=(n_q // qblk,),
        in_specs=[
            pl.BlockSpec((qblk, d), lambda j: (j, 0)),
            pl.BlockSpec((d, qblk * n_cand), lambda j: (0, j)),
            pl.BlockSpec((qblk, 1), lambda j: (j, 0)),
            pl.BlockSpec((qblk, 1), lambda j: (j, 0)),
        ],
        out_specs=[
            pl.BlockSpec((qblk, 1), lambda j: (j, 0)),
            pl.BlockSpec((qblk, 1), lambda j: (j, 0)),
        ],
        out_shape=[
            jax.ShapeDtypeStruct((n_q, 1), jnp.int32),
            jax.ShapeDtypeStruct((n_q, 1), jnp.int32),
        ],
    )(doc_bf, gath, vmax, base)

    need_fallback = (jnp.max(cnt) > 1) | (jnp.min(ok) < 1)

    def _slow(_):
        fb_block = 2048
        return _full_kernel(doc_bf, cls_bf, n_keys, d, n_q, fb_block,
                            pl.cdiv(n_keys, fb_block))

    def _fast(_):
        return vmax[:, 0], idx_fast[:, 0]

    return jax.lax.cond(need_fallback, _slow, _fast, None)
